# baseline (speedup 1.0000x reference)
import functools

import jax
import jax.numpy as jnp
from jax import lax
from jax.experimental import pallas as pl
from jax.experimental.pallas import tpu as pltpu
from jax.experimental.pallas import tpu_sc as plsc

K = 8192
CODE_DIM = 256
B = 16
HW = 1024
N = B * HW


TM = 512
GRID = N // TM


_SEG = (0, 2816, 5632, 8192)


def _argmax_body(x_ref, w_ref, idx_ref):
    x = x_ref[...].astype(jnp.bfloat16)
    w = w_ref[...].astype(jnp.bfloat16)
    logits = lax.dot_general(
        x, w, (((1,), (1,)), ((), ())),
        preferred_element_type=jnp.float32,
    )

    lane = lax.broadcasted_iota(jnp.int32, (TM, 128), 1)

    def seg_max_arg(lo, hi):
        seg = logits[:, lo:hi]
        m = jnp.max(seg, axis=1, keepdims=True)
        nch = (hi - lo) // 128
        macc = jnp.full((TM, 128), K, jnp.int32)
        for c in reversed(range(nch)):
            eq = seg[:, c * 128:(c + 1) * 128] == m
            macc = jnp.where(eq, jnp.int32(c), macc)
        kk = jnp.where(macc == K, jnp.int32(K), macc * 128 + lane)
        idx = jnp.min(kk, axis=1) + lo
        return m[:, 0], idx

    v0, i0 = seg_max_arg(_SEG[0], _SEG[1])
    v1, i1 = seg_max_arg(_SEG[1], _SEG[2])
    v2, i2 = seg_max_arg(_SEG[2], _SEG[3])
    acc_v = v0.astype(jnp.bfloat16).astype(jnp.float32)
    acc_i = i0
    w1 = v1 > acc_v
    acc_v = jnp.where(w1, v1.astype(jnp.bfloat16).astype(jnp.float32), acc_v)
    acc_i = jnp.where(w1, i1, acc_i)
    w2 = v2 > acc_v
    acc_i = jnp.where(w2, i2, acc_i)
    idx_ref[0, 0, :] = acc_i


def _compute_indices(n):
    grid = n // TM
    return pl.pallas_call(
        _argmax_body,
        grid=(grid,),
        in_specs=[
            pl.BlockSpec((TM, CODE_DIM), lambda i: (i, 0)),
            pl.BlockSpec((K, CODE_DIM), lambda i: (0, 0)),
        ],
        out_specs=pl.BlockSpec((1, 1, TM), lambda i: (i, 0, 0)),
        out_shape=jax.ShapeDtypeStruct((grid, 1, TM), jnp.int32),
    )


NC, NS = 2, 16
NW = NC * NS
B_PER_W = N // NW
CHUNK = 128
NCHUNK = B_PER_W // CHUNK


@functools.cache
def _gather(n):
    b_per_w = n // NW
    nchunk = b_per_w // CHUNK

    def body(idx_hbm, table_hbm, out_hbm, idx_v, rows_v, sem):
        wid = lax.axis_index("s") * NC + lax.axis_index("c")
        base = wid * b_per_w
        for i in range(nchunk):
            off = base + i * CHUNK
            pltpu.sync_copy(idx_hbm.at[pl.ds(off, CHUNK)], idx_v)
            pltpu.async_copy(table_hbm.at[idx_v], rows_v, sem).wait()
            pltpu.sync_copy(rows_v, out_hbm.at[pl.ds(off, CHUNK)])

    return pl.kernel(
        body,
        out_type=jax.ShapeDtypeStruct((n, CODE_DIM), jnp.float32),
        mesh=plsc.VectorSubcoreMesh(core_axis_name="c", subcore_axis_name="s"),
        scratch_types=[
            pltpu.VMEM((CHUNK,), jnp.int32),
            pltpu.VMEM((CHUNK, CODE_DIM), jnp.float32),
            pltpu.SemaphoreType.DMA,
        ],
    )


def kernel(inputs, W):
    x = inputs.reshape(N, CODE_DIM)
    h = N // 2
    halves = []
    for p in range(2):
        xp = lax.slice_in_dim(x, p * h, (p + 1) * h, axis=0)
        idx = _compute_indices(h)(xp, W).reshape(h)
        halves.append(_gather(h)(idx, W))
    return jnp.concatenate(halves, axis=0).reshape(B, HW, CODE_DIM)


# speedup vs baseline: 1.4349x; 1.4349x over previous
import functools

import jax
import jax.numpy as jnp
from jax import lax
from jax.experimental import pallas as pl
from jax.experimental.pallas import tpu as pltpu
from jax.experimental.pallas import tpu_sc as plsc

K = 8192
CODE_DIM = 256
B = 16
HW = 1024
N = B * HW


TM = 512
GRID = N // TM


_SEG = (0, 2816, 5632, 8192)


def _argmax_body(x_ref, w_ref, idx_ref):
    x = x_ref[...].astype(jnp.bfloat16)
    w = w_ref[...].astype(jnp.bfloat16)
    logits = lax.dot_general(
        x, w, (((1,), (1,)), ((), ())),
        preferred_element_type=jnp.float32,
    )

    lane = lax.broadcasted_iota(jnp.int32, (TM, 128), 1)

    def seg_max_arg(lo, hi):
        seg = logits[:, lo:hi]
        nch = (hi - lo) // 128
        acc_v = seg[:, 0:128]
        acc_c = jnp.zeros((TM, 128), jnp.int32)
        for c in range(1, nch):
            v = seg[:, c * 128:(c + 1) * 128]
            gt = v > acc_v
            acc_v = jnp.where(gt, v, acc_v)
            acc_c = jnp.where(gt, jnp.int32(c), acc_c)
        m = jnp.max(acc_v, axis=1, keepdims=True)
        kk = jnp.where(acc_v == m, acc_c * 128 + lane + lo, jnp.int32(K))
        idx = jnp.min(kk, axis=1)
        return m[:, 0], idx

    v0, i0 = seg_max_arg(_SEG[0], _SEG[1])
    v1, i1 = seg_max_arg(_SEG[1], _SEG[2])
    v2, i2 = seg_max_arg(_SEG[2], _SEG[3])
    acc_v = v0.astype(jnp.bfloat16).astype(jnp.float32)
    acc_i = i0
    w1 = v1 > acc_v
    acc_v = jnp.where(w1, v1.astype(jnp.bfloat16).astype(jnp.float32), acc_v)
    acc_i = jnp.where(w1, i1, acc_i)
    w2 = v2 > acc_v
    acc_i = jnp.where(w2, i2, acc_i)
    idx_ref[0, 0, :] = acc_i


def _compute_indices(n):
    grid = n // TM
    return pl.pallas_call(
        _argmax_body,
        grid=(grid,),
        in_specs=[
            pl.BlockSpec((TM, CODE_DIM), lambda i: (i, 0)),
            pl.BlockSpec((K, CODE_DIM), lambda i: (0, 0)),
        ],
        out_specs=pl.BlockSpec((1, 1, TM), lambda i: (i, 0, 0)),
        out_shape=jax.ShapeDtypeStruct((grid, 1, TM), jnp.int32),
    )


NC, NS = 2, 16
NW = NC * NS
B_PER_W = N // NW
CHUNK = 128
NCHUNK = B_PER_W // CHUNK


@functools.cache
def _gather(n):
    b_per_w = n // NW
    nchunk = b_per_w // CHUNK

    def body(idx_hbm, table_hbm, out_hbm, idx_v, rows_v, sem):
        wid = lax.axis_index("s") * NC + lax.axis_index("c")
        base = wid * b_per_w
        for i in range(nchunk):
            off = base + i * CHUNK
            pltpu.sync_copy(idx_hbm.at[pl.ds(off, CHUNK)], idx_v)
            pltpu.async_copy(table_hbm.at[idx_v], rows_v, sem).wait()
            pltpu.sync_copy(rows_v, out_hbm.at[pl.ds(off, CHUNK)])

    return pl.kernel(
        body,
        out_type=jax.ShapeDtypeStruct((n, CODE_DIM), jnp.float32),
        mesh=plsc.VectorSubcoreMesh(core_axis_name="c", subcore_axis_name="s"),
        scratch_types=[
            pltpu.VMEM((CHUNK,), jnp.int32),
            pltpu.VMEM((CHUNK, CODE_DIM), jnp.float32),
            pltpu.SemaphoreType.DMA,
        ],
    )


def kernel(inputs, W):
    x = inputs.reshape(N, CODE_DIM)
    idx = _compute_indices(N)(x, W).reshape(N)
    embed = _gather(N)(idx, W)
    return embed.reshape(B, HW, CODE_DIM)


# speedup vs baseline: 1.4566x; 1.0151x over previous
import functools

import jax
import jax.numpy as jnp
from jax import lax
from jax.experimental import pallas as pl
from jax.experimental.pallas import tpu as pltpu
from jax.experimental.pallas import tpu_sc as plsc

K = 8192
CODE_DIM = 256
B = 16
HW = 1024
N = B * HW


TM = 1024
GRID = N // TM


_SEG = (0, 2816, 5632, 8192)


def _argmax_body(x_ref, w_ref, idx_ref):
    x = x_ref[...].astype(jnp.bfloat16)
    w = w_ref[...].astype(jnp.bfloat16)
    logits = lax.dot_general(
        x, w, (((1,), (1,)), ((), ())),
        preferred_element_type=jnp.float32,
    )

    lane = lax.broadcasted_iota(jnp.int32, (TM, 128), 1)

    def seg_max_arg(lo, hi):
        seg = logits[:, lo:hi]
        nch = (hi - lo) // 128
        acc_v = seg[:, 0:128]
        acc_c = jnp.zeros((TM, 128), jnp.int32)
        for c in range(1, nch):
            v = seg[:, c * 128:(c + 1) * 128]
            gt = v > acc_v
            acc_v = jnp.where(gt, v, acc_v)
            acc_c = jnp.where(gt, jnp.int32(c), acc_c)
        m = jnp.max(acc_v, axis=1, keepdims=True)
        kk = jnp.where(acc_v == m, acc_c * 128 + lane + lo, jnp.int32(K))
        idx = jnp.min(kk, axis=1)
        return m[:, 0], idx

    v0, i0 = seg_max_arg(_SEG[0], _SEG[1])
    v1, i1 = seg_max_arg(_SEG[1], _SEG[2])
    v2, i2 = seg_max_arg(_SEG[2], _SEG[3])
    acc_v = v0.astype(jnp.bfloat16).astype(jnp.float32)
    acc_i = i0
    w1 = v1 > acc_v
    acc_v = jnp.where(w1, v1.astype(jnp.bfloat16).astype(jnp.float32), acc_v)
    acc_i = jnp.where(w1, i1, acc_i)
    w2 = v2 > acc_v
    acc_i = jnp.where(w2, i2, acc_i)
    idx_ref[0, 0, :] = acc_i


def _compute_indices(n):
    grid = n // TM
    return pl.pallas_call(
        _argmax_body,
        grid=(grid,),
        in_specs=[
            pl.BlockSpec((TM, CODE_DIM), lambda i: (i, 0)),
            pl.BlockSpec((K, CODE_DIM), lambda i: (0, 0)),
        ],
        out_specs=pl.BlockSpec((1, 1, TM), lambda i: (i, 0, 0)),
        out_shape=jax.ShapeDtypeStruct((grid, 1, TM), jnp.int32),
    )


NC, NS = 2, 16
NW = NC * NS
B_PER_W = N // NW
CHUNK = 128
NCHUNK = B_PER_W // CHUNK


@functools.cache
def _gather(n):
    b_per_w = n // NW
    nchunk = b_per_w // CHUNK

    def body(idx_hbm, table_hbm, out_hbm, idx_v, rows_v, sem):
        wid = lax.axis_index("s") * NC + lax.axis_index("c")
        base = wid * b_per_w
        for i in range(nchunk):
            off = base + i * CHUNK
            pltpu.sync_copy(idx_hbm.at[pl.ds(off, CHUNK)], idx_v)
            pltpu.async_copy(table_hbm.at[idx_v], rows_v, sem).wait()
            pltpu.sync_copy(rows_v, out_hbm.at[pl.ds(off, CHUNK)])

    return pl.kernel(
        body,
        out_type=jax.ShapeDtypeStruct((n, CODE_DIM), jnp.float32),
        mesh=plsc.VectorSubcoreMesh(core_axis_name="c", subcore_axis_name="s"),
        scratch_types=[
            pltpu.VMEM((CHUNK,), jnp.int32),
            pltpu.VMEM((CHUNK, CODE_DIM), jnp.float32),
            pltpu.SemaphoreType.DMA,
        ],
    )


def kernel(inputs, W):
    x = inputs.reshape(N, CODE_DIM)
    idx = _compute_indices(N)(x, W).reshape(N)
    embed = _gather(N)(idx, W)
    return embed.reshape(B, HW, CODE_DIM)


# speedup vs baseline: 1.4756x; 1.0131x over previous
import functools

import jax
import jax.numpy as jnp
from jax import lax
from jax.experimental import pallas as pl
from jax.experimental.pallas import tpu as pltpu
from jax.experimental.pallas import tpu_sc as plsc

K = 8192
CODE_DIM = 256
B = 16
HW = 1024
N = B * HW


TM = 1024
GRID = N // TM


_SEG = (0, 2816, 5632, 8192)


def _argmax_body(x_ref, w_ref, idx_ref):
    x = x_ref[...].astype(jnp.bfloat16)
    w = w_ref[...].astype(jnp.bfloat16)
    logits = lax.dot_general(
        x, w, (((1,), (1,)), ((), ())),
        preferred_element_type=jnp.float32,
    )

    lane = lax.broadcasted_iota(jnp.int32, (TM, 128), 1)

    def seg_max_arg(lo, hi):
        seg = logits[:, lo:hi]
        nch = (hi - lo) // 128
        acc_v = seg[:, 0:128]
        acc_c = jnp.zeros((TM, 128), jnp.int32)
        for c in range(1, nch):
            v = seg[:, c * 128:(c + 1) * 128]
            gt = v > acc_v
            acc_v = jnp.where(gt, v, acc_v)
            acc_c = jnp.where(gt, jnp.int32(c), acc_c)
        m = jnp.max(acc_v, axis=1, keepdims=True)
        kk = jnp.where(acc_v == m, acc_c * 128 + lane + lo, jnp.int32(K))
        idx = jnp.min(kk, axis=1)
        return m[:, 0], idx

    v0, i0 = seg_max_arg(_SEG[0], _SEG[1])
    v1, i1 = seg_max_arg(_SEG[1], _SEG[2])
    v2, i2 = seg_max_arg(_SEG[2], _SEG[3])
    acc_v = v0.astype(jnp.bfloat16).astype(jnp.float32)
    acc_i = i0
    w1 = v1 > acc_v
    acc_v = jnp.where(w1, v1.astype(jnp.bfloat16).astype(jnp.float32), acc_v)
    acc_i = jnp.where(w1, i1, acc_i)
    w2 = v2 > acc_v
    acc_i = jnp.where(w2, i2, acc_i)
    idx_ref[0, 0, :] = acc_i


def _compute_indices(n):
    grid = n // TM
    return pl.pallas_call(
        _argmax_body,
        grid=(grid,),
        in_specs=[
            pl.BlockSpec((TM, CODE_DIM), lambda i: (i, 0)),
            pl.BlockSpec((K, CODE_DIM), lambda i: (0, 0)),
        ],
        out_specs=pl.BlockSpec((1, 1, TM), lambda i: (i, 0, 0)),
        out_shape=jax.ShapeDtypeStruct((grid, 1, TM), jnp.int32),
    )


NC, NS = 2, 16
NW = NC * NS
B_PER_W = N // NW
CHUNK = 128
NCHUNK = B_PER_W // CHUNK


@functools.cache
def _gather(n):
    b_per_w = n // NW
    nchunk = b_per_w // CHUNK

    def body(idx_hbm, table_hbm, out_hbm,
             idx_v0, idx_v1, rows_v0, rows_v1, gs0, gs1, ss0, ss1):
        wid = lax.axis_index("s") * NC + lax.axis_index("c")
        base = wid * b_per_w
        idx_v = (idx_v0, idx_v1)
        rows_v = (rows_v0, rows_v1)
        gs = (gs0, gs1)
        ss = (ss0, ss1)
        pltpu.sync_copy(idx_hbm.at[pl.ds(base, CHUNK)], idx_v0)
        g0 = pltpu.async_copy(table_hbm.at[idx_v0], rows_v0, gs0)
        scat = [None, None]
        gath = [g0, None]
        for i in range(nchunk):
            b = i % 2
            nb = (i + 1) % 2
            if i + 1 < nchunk:
                off = base + (i + 1) * CHUNK
                if scat[nb] is not None:
                    scat[nb].wait()
                    scat[nb] = None
                pltpu.sync_copy(idx_hbm.at[pl.ds(off, CHUNK)], idx_v[nb])
                gath[nb] = pltpu.async_copy(table_hbm.at[idx_v[nb]], rows_v[nb], gs[nb])
            gath[b].wait()
            off = base + i * CHUNK
            scat[b] = pltpu.async_copy(rows_v[b], out_hbm.at[pl.ds(off, CHUNK)], ss[b])
        for s in scat:
            if s is not None:
                s.wait()

    return pl.kernel(
        body,
        out_type=jax.ShapeDtypeStruct((n, CODE_DIM), jnp.float32),
        mesh=plsc.VectorSubcoreMesh(core_axis_name="c", subcore_axis_name="s"),
        scratch_types=[
            pltpu.VMEM((CHUNK,), jnp.int32),
            pltpu.VMEM((CHUNK,), jnp.int32),
            pltpu.VMEM((CHUNK, CODE_DIM), jnp.float32),
            pltpu.VMEM((CHUNK, CODE_DIM), jnp.float32),
            pltpu.SemaphoreType.DMA,
            pltpu.SemaphoreType.DMA,
            pltpu.SemaphoreType.DMA,
            pltpu.SemaphoreType.DMA,
        ],
    )


def kernel(inputs, W):
    x = inputs.reshape(N, CODE_DIM)
    idx = _compute_indices(N)(x, W).reshape(N)
    embed = _gather(N)(idx, W)
    return embed.reshape(B, HW, CODE_DIM)
